# Initial kernel scaffold; baseline (speedup 1.0000x reference)
import jax
import jax.numpy as jnp
from jax.experimental import pallas as pl


def kernel(x, edge_index_0, edge_index_1, edge_index_2, W_l_0, b_l_0, W_r_0, W_l_1, b_l_1, W_r_1, W_l_2, b_l_2, W_r_2, W_l_f, b_l_f, W_r_f):
    raise NotImplementedError("write your pallas kernel here")


# speedup vs baseline: 2.9004x; 2.9004x over previous
import jax
import jax.numpy as jnp
import numpy as np
from jax import lax
from jax.experimental import pallas as pl
from jax.experimental.pallas import tpu as pltpu
from jax.experimental.pallas import tpu_sc as plsc

N = 10000
D = 128
E = 320000
R = 3

NC = 2
NS = 16
NW = NC * NS
BATCH = 128
NB = 80
E_PAD = NW * NB * BATCH
TILE_ROWS = 632
N_PAD = NS * TILE_ROWS
HR = 80
BLK = 1024
GRID = 10


def _pad_edges(ei):
    src = ei[0].astype(jnp.int32)
    dst = ei[1].astype(jnp.int32)
    pad = E_PAD - E
    src = jnp.concatenate([src, jnp.zeros((pad,), jnp.int32)])
    dst = jnp.concatenate([dst, jnp.full((pad,), N, jnp.int32)])
    return src.reshape(-1, BATCH), dst.reshape(-1, BATCH)


def _make_seg(num_rel, with_deg):
    mesh = plsc.VectorSubcoreMesh(core_axis_name="c", subcore_axis_name="s")
    out_type = [jax.ShapeDtypeStruct((num_rel * NC * N_PAD, D), jnp.float32)]
    if with_deg:
        out_type.append(jax.ShapeDtypeStruct((num_rel * NW * HR, D), jnp.float32))
    scratch = [
        pltpu.VMEM((8, BATCH), jnp.int32),
        pltpu.VMEM((8, BATCH), jnp.int32),
        pltpu.VMEM((BATCH, D), jnp.float32),
        pltpu.VMEM((HR, D), jnp.float32),
        pltpu.VMEM_SHARED((N_PAD, D), jnp.float32),
        pltpu.SemaphoreType.DMA,
    ]

    def body(table_hbm, src_hbm, dst_hbm, z128_hbm, *out_and_scratch):
        if with_deg:
            acc_out, deg_out = out_and_scratch[:2]
            rest = out_and_scratch[2:]
        else:
            acc_out, = out_and_scratch[:1]
            rest = out_and_scratch[1:]
        src_v, dst_v, rows_v, hist_v, acc_s, sem = rest

        cid = lax.axis_index("c")
        sid = lax.axis_index("s")
        w = cid * NS + sid
        row0 = sid * TILE_ROWS
        ones16 = jnp.ones((16,), jnp.float32)

        for r in range(num_rel):
            pltpu.sync_copy(z128_hbm, acc_s.at[pl.ds(row0, TILE_ROWS)])
            if with_deg:
                pltpu.sync_copy(z128_hbm.at[pl.ds(0, HR)], hist_v)
            ebase = (r * NW + w) * NB
            plsc.subcore_barrier()

            def group_body(g, carry):
                pltpu.sync_copy(src_hbm.at[pl.ds(ebase + g * 8, 8)], src_v)
                pltpu.sync_copy(dst_hbm.at[pl.ds(ebase + g * 8, 8)], dst_v)
                for jj in range(8):
                    pltpu.async_copy(table_hbm.at[src_v.at[jj]], rows_v, sem).wait()
                    pltpu.sync_copy(rows_v, acc_s.at[dst_v.at[jj]], add=True)
                    if with_deg:
                        for k in range(BATCH // 16):
                            idx = dst_v[jj, pl.ds(k * 16, 16)]
                            plsc.addupdate_scatter(
                                hist_v,
                                [lax.shift_right_logical(idx, 7),
                                 jnp.bitwise_and(idx, 127)],
                                ones16)
                return carry

            lax.fori_loop(0, NB // 8, group_body, 0)
            plsc.subcore_barrier()

            obase = (r * NC + cid) * N_PAD + row0
            pltpu.sync_copy(acc_s.at[pl.ds(row0, TILE_ROWS)],
                            acc_out.at[pl.ds(obase, TILE_ROWS)])
            if with_deg:
                pltpu.sync_copy(hist_v, deg_out.at[pl.ds((r * NW + w) * HR, HR)])

    return pl.kernel(
        body, out_type=out_type, mesh=mesh, scratch_types=scratch,
        compiler_params=pltpu.CompilerParams(needs_layout_passes=False),
        name=f"sc_segsum_{num_rel}")


_seg3 = _make_seg(R, True)
_seg1 = _make_seg(1, False)

_A2_np = (np.arange(BLK)[:, None] // D == np.arange(NW * 8)[None, :] % 8)
_M_np = (np.arange(BLK)[:, None] % D == np.arange(D)[None, :])


def _pernode_deg(deg_block, a2, m):
    rowvals = jnp.dot(a2, deg_block.reshape(NW * 8, D),
                      preferred_element_type=jnp.float32)
    return jnp.sum(rowvals * m, axis=1, keepdims=True)


def _tc1_body(acc_ref, deg_ref, x_ref, a2_ref, m_ref, wl_ref, bl_ref, wr_ref,
              h_ref):
    x = x_ref[...]
    a2 = a2_ref[...]
    m = m_ref[...]
    total = jnp.zeros((BLK, D), jnp.float32)
    for r in range(R):
        agg = acc_ref[r, 0] + acc_ref[r, 1]
        deg = _pernode_deg(deg_ref[r], a2, m)
        agg = agg / jnp.maximum(deg, 1.0)
        y = (jnp.dot(agg, wl_ref[r], preferred_element_type=jnp.float32)
             + bl_ref[r, 0]
             + jnp.dot(x, wr_ref[r], preferred_element_type=jnp.float32))
        total = total + jnp.maximum(y, 0.0)
    h_ref[...] = total * (1.0 / R)


def _tc2_body(acc_ref, deg_ref, h_ref, a2_ref, m_ref, wl_ref, bl_ref, wr_ref,
              out_ref):
    h = h_ref[...]
    agg = acc_ref[0] + acc_ref[1]
    deg = _pernode_deg(deg_ref[...], a2_ref[...], m_ref[...])
    agg = agg / jnp.maximum(deg, 1.0)
    out_ref[...] = (jnp.dot(agg, wl_ref[...], preferred_element_type=jnp.float32)
                    + bl_ref[0]
                    + jnp.dot(h, wr_ref[...], preferred_element_type=jnp.float32))


def kernel(x, edge_index_0, edge_index_1, edge_index_2,
           W_l_0, b_l_0, W_r_0,
           W_l_1, b_l_1, W_r_1,
           W_l_2, b_l_2, W_r_2,
           W_l_f, b_l_f, W_r_f):
    pads = [_pad_edges(e) for e in (edge_index_0, edge_index_1, edge_index_2)]
    srcs = jnp.concatenate([p[0] for p in pads])
    dsts = jnp.concatenate([p[1] for p in pads])
    z128 = jnp.zeros((TILE_ROWS, D), jnp.float32)
    a2 = jnp.asarray(_A2_np, jnp.float32)
    m = jnp.asarray(_M_np, jnp.float32)

    acc, deg = _seg3(x, srcs, dsts, z128)
    acc = acc.reshape(R, NC, N_PAD, D)
    deg = deg.reshape(R, NW, HR, D)

    wl = jnp.stack([W_l_0, W_l_1, W_l_2])
    bl = jnp.stack([b_l_0, b_l_1, b_l_2]).reshape(R, 1, D)
    wr = jnp.stack([W_r_0, W_r_1, W_r_2])

    h = pl.pallas_call(
        _tc1_body,
        grid=(GRID,),
        in_specs=[
            pl.BlockSpec((R, NC, BLK, D), lambda i: (0, 0, i, 0)),
            pl.BlockSpec((R, NW, 8, D), lambda i: (0, 0, i, 0)),
            pl.BlockSpec((BLK, D), lambda i: (i, 0)),
            pl.BlockSpec((BLK, NW * 8), lambda i: (0, 0)),
            pl.BlockSpec((BLK, D), lambda i: (0, 0)),
            pl.BlockSpec((R, D, D), lambda i: (0, 0, 0)),
            pl.BlockSpec((R, 1, D), lambda i: (0, 0, 0)),
            pl.BlockSpec((R, D, D), lambda i: (0, 0, 0)),
        ],
        out_specs=pl.BlockSpec((BLK, D), lambda i: (i, 0)),
        out_shape=jax.ShapeDtypeStruct((N, D), jnp.float32),
    )(acc, deg, x, a2, m, wl, bl, wr)

    accf, = _seg1(h, srcs[:NW * NB], dsts[:NW * NB], z128)
    accf = accf.reshape(NC, N_PAD, D)

    out = pl.pallas_call(
        _tc2_body,
        grid=(GRID,),
        in_specs=[
            pl.BlockSpec((NC, BLK, D), lambda i: (0, i, 0)),
            pl.BlockSpec((NW, 8, D), lambda i: (0, i, 0)),
            pl.BlockSpec((BLK, D), lambda i: (i, 0)),
            pl.BlockSpec((BLK, NW * 8), lambda i: (0, 0)),
            pl.BlockSpec((BLK, D), lambda i: (0, 0)),
            pl.BlockSpec((D, D), lambda i: (0, 0)),
            pl.BlockSpec((1, D), lambda i: (0, 0)),
            pl.BlockSpec((D, D), lambda i: (0, 0)),
        ],
        out_specs=pl.BlockSpec((BLK, D), lambda i: (i, 0)),
        out_shape=jax.ShapeDtypeStruct((N, D), jnp.float32),
    )(accf, deg[0], h, a2, m, W_l_f, b_l_f.reshape(1, D), W_r_f)

    return out


# speedup vs baseline: 3.2379x; 1.1163x over previous
import jax
import jax.numpy as jnp
import numpy as np
from jax import lax
from jax.experimental import pallas as pl
from jax.experimental.pallas import tpu as pltpu
from jax.experimental.pallas import tpu_sc as plsc

N = 10000
D = 128
E = 320000
R = 3

NC = 2
NS = 16
NW = NC * NS
BATCH = 128
NB = 80
E_PAD = NW * NB * BATCH
TILE_ROWS = 632
N_PAD = NS * TILE_ROWS
HR = 80
BLK = 1024
GRID = 10


NG = NB // 8


def _pad_edges(ei):
    src = ei[0].astype(jnp.int32)
    dst = ei[1].astype(jnp.int32)
    pad = E_PAD - E
    src = jnp.concatenate([src, jnp.zeros((pad,), jnp.int32)])
    dst = jnp.concatenate([dst, jnp.full((pad,), N, jnp.int32)])
    src = src.reshape(NW, NG, 8, BATCH)
    dst = dst.reshape(NW, NG, 8, BATCH)
    return jnp.concatenate([src, dst], axis=2).reshape(-1, BATCH)


def _make_seg(num_rel, with_deg):
    mesh = plsc.VectorSubcoreMesh(core_axis_name="c", subcore_axis_name="s")
    out_type = [jax.ShapeDtypeStruct((num_rel * NC * N_PAD, D), jnp.float32)]
    if with_deg:
        out_type.append(jax.ShapeDtypeStruct((num_rel * NW * HR, D), jnp.float32))
    scratch = [
        pltpu.VMEM((16, BATCH), jnp.int32),
        pltpu.VMEM((2, BATCH, D), jnp.float32),
        pltpu.VMEM((HR, D), jnp.float32),
        pltpu.VMEM_SHARED((N_PAD, D), jnp.float32),
        pltpu.SemaphoreType.DMA,
        pltpu.SemaphoreType.DMA,
        pltpu.SemaphoreType.DMA,
        pltpu.SemaphoreType.DMA,
    ]

    def body(table_hbm, sd_hbm, z128_hbm, *out_and_scratch):
        if with_deg:
            acc_out, deg_out = out_and_scratch[:2]
            rest = out_and_scratch[2:]
        else:
            acc_out, = out_and_scratch[:1]
            rest = out_and_scratch[1:]
        sd_v, rows_v, hist_v, acc_s, gsem0, gsem1, ssem0, ssem1 = rest
        gsems = (gsem0, gsem1)
        ssems = (ssem0, ssem1)

        cid = lax.axis_index("c")
        sid = lax.axis_index("s")
        w = cid * NS + sid
        row0 = sid * TILE_ROWS
        ones16 = jnp.ones((16,), jnp.float32)

        def drain_scatter(b):
            pltpu.make_async_copy(rows_v.at[b], acc_s.at[sd_v.at[8]],
                                  ssems[b]).wait()

        for r in range(num_rel):
            pltpu.sync_copy(z128_hbm, acc_s.at[pl.ds(row0, TILE_ROWS)])
            if with_deg:
                pltpu.sync_copy(z128_hbm.at[pl.ds(0, HR)], hist_v)
            plsc.subcore_barrier()
            gbase0 = (r * NW + w) * NG * 16
            pltpu.sync_copy(sd_hbm.at[pl.ds(gbase0, 16)], sd_v)
            pltpu.sync_copy(z128_hbm.at[pl.ds(0, BATCH)], rows_v.at[0])
            pltpu.sync_copy(z128_hbm.at[pl.ds(0, BATCH)], rows_v.at[1])
            pltpu.async_copy(rows_v.at[0], acc_s.at[sd_v.at[8]], ssem0,
                             add=True)
            pltpu.async_copy(rows_v.at[1], acc_s.at[sd_v.at[8]], ssem1,
                             add=True)

            def group_body(g, carry):
                drain_scatter(0)
                drain_scatter(1)
                gbase = ((r * NW + w) * NG + g) * 16
                pltpu.sync_copy(sd_hbm.at[pl.ds(gbase, 16)], sd_v)
                descs = {
                    0: pltpu.async_copy(table_hbm.at[sd_v.at[0]],
                                        rows_v.at[0], gsem0),
                    1: pltpu.async_copy(table_hbm.at[sd_v.at[1]],
                                        rows_v.at[1], gsem1),
                }
                for jj in range(8):
                    b = jj & 1
                    descs[jj].wait()
                    pltpu.async_copy(rows_v.at[b],
                                     acc_s.at[sd_v.at[8 + jj]], ssems[b],
                                     add=True)
                    if with_deg:
                        for k in range(BATCH // 16):
                            idx = sd_v[8 + jj, pl.ds(k * 16, 16)]
                            plsc.addupdate_scatter(
                                hist_v,
                                [lax.shift_right_logical(idx, 7),
                                 jnp.bitwise_and(idx, 127)],
                                ones16)
                    if jj < 6:
                        drain_scatter(b)
                        descs[jj + 2] = pltpu.async_copy(
                            table_hbm.at[sd_v.at[jj + 2]], rows_v.at[b],
                            gsems[b])
                return carry

            lax.fori_loop(0, NG, group_body, 0)
            drain_scatter(0)
            drain_scatter(1)
            plsc.subcore_barrier()

            obase = (r * NC + cid) * N_PAD + row0
            pltpu.sync_copy(acc_s.at[pl.ds(row0, TILE_ROWS)],
                            acc_out.at[pl.ds(obase, TILE_ROWS)])
            if with_deg:
                pltpu.sync_copy(hist_v, deg_out.at[pl.ds((r * NW + w) * HR, HR)])

    return pl.kernel(
        body, out_type=out_type, mesh=mesh, scratch_types=scratch,
        compiler_params=pltpu.CompilerParams(needs_layout_passes=False),
        name=f"sc_segsum_{num_rel}")


_seg3 = _make_seg(R, True)
_seg1 = _make_seg(1, False)

_A2_np = (np.arange(BLK)[:, None] // D == np.arange(NW * 8)[None, :] % 8)
_M_np = (np.arange(BLK)[:, None] % D == np.arange(D)[None, :])


def _pernode_deg(deg_block, a2, m):
    rowvals = jnp.dot(a2, deg_block.reshape(NW * 8, D),
                      preferred_element_type=jnp.float32)
    return jnp.sum(rowvals * m, axis=1, keepdims=True)


def _tc1_body(acc_ref, deg_ref, x_ref, a2_ref, m_ref, wl_ref, bl_ref, wr_ref,
              h_ref):
    x = x_ref[...]
    a2 = a2_ref[...]
    m = m_ref[...]
    total = jnp.zeros((BLK, D), jnp.float32)
    for r in range(R):
        agg = acc_ref[r, 0] + acc_ref[r, 1]
        deg = _pernode_deg(deg_ref[r], a2, m)
        agg = agg / jnp.maximum(deg, 1.0)
        y = (jnp.dot(agg, wl_ref[r], preferred_element_type=jnp.float32)
             + bl_ref[r, 0]
             + jnp.dot(x, wr_ref[r], preferred_element_type=jnp.float32))
        total = total + jnp.maximum(y, 0.0)
    h_ref[...] = total * (1.0 / R)


def _tc2_body(acc_ref, deg_ref, h_ref, a2_ref, m_ref, wl_ref, bl_ref, wr_ref,
              out_ref):
    h = h_ref[...]
    agg = acc_ref[0] + acc_ref[1]
    deg = _pernode_deg(deg_ref[...], a2_ref[...], m_ref[...])
    agg = agg / jnp.maximum(deg, 1.0)
    out_ref[...] = (jnp.dot(agg, wl_ref[...], preferred_element_type=jnp.float32)
                    + bl_ref[0]
                    + jnp.dot(h, wr_ref[...], preferred_element_type=jnp.float32))


def kernel(x, edge_index_0, edge_index_1, edge_index_2,
           W_l_0, b_l_0, W_r_0,
           W_l_1, b_l_1, W_r_1,
           W_l_2, b_l_2, W_r_2,
           W_l_f, b_l_f, W_r_f):
    sds = jnp.concatenate(
        [_pad_edges(e) for e in (edge_index_0, edge_index_1, edge_index_2)])
    z128 = jnp.zeros((TILE_ROWS, D), jnp.float32)
    a2 = jnp.asarray(_A2_np, jnp.float32)
    m = jnp.asarray(_M_np, jnp.float32)

    acc, deg = _seg3(x, sds, z128)
    acc = acc.reshape(R, NC, N_PAD, D)
    deg = deg.reshape(R, NW, HR, D)

    wl = jnp.stack([W_l_0, W_l_1, W_l_2])
    bl = jnp.stack([b_l_0, b_l_1, b_l_2]).reshape(R, 1, D)
    wr = jnp.stack([W_r_0, W_r_1, W_r_2])

    h = pl.pallas_call(
        _tc1_body,
        grid=(GRID,),
        in_specs=[
            pl.BlockSpec((R, NC, BLK, D), lambda i: (0, 0, i, 0)),
            pl.BlockSpec((R, NW, 8, D), lambda i: (0, 0, i, 0)),
            pl.BlockSpec((BLK, D), lambda i: (i, 0)),
            pl.BlockSpec((BLK, NW * 8), lambda i: (0, 0)),
            pl.BlockSpec((BLK, D), lambda i: (0, 0)),
            pl.BlockSpec((R, D, D), lambda i: (0, 0, 0)),
            pl.BlockSpec((R, 1, D), lambda i: (0, 0, 0)),
            pl.BlockSpec((R, D, D), lambda i: (0, 0, 0)),
        ],
        out_specs=pl.BlockSpec((BLK, D), lambda i: (i, 0)),
        out_shape=jax.ShapeDtypeStruct((N, D), jnp.float32),
    )(acc, deg, x, a2, m, wl, bl, wr)

    accf, = _seg1(h, sds[:NW * NG * 16], z128)
    accf = accf.reshape(NC, N_PAD, D)

    out = pl.pallas_call(
        _tc2_body,
        grid=(GRID,),
        in_specs=[
            pl.BlockSpec((NC, BLK, D), lambda i: (0, i, 0)),
            pl.BlockSpec((NW, 8, D), lambda i: (0, i, 0)),
            pl.BlockSpec((BLK, D), lambda i: (i, 0)),
            pl.BlockSpec((BLK, NW * 8), lambda i: (0, 0)),
            pl.BlockSpec((BLK, D), lambda i: (0, 0)),
            pl.BlockSpec((D, D), lambda i: (0, 0)),
            pl.BlockSpec((1, D), lambda i: (0, 0)),
            pl.BlockSpec((D, D), lambda i: (0, 0)),
        ],
        out_specs=pl.BlockSpec((BLK, D), lambda i: (i, 0)),
        out_shape=jax.ShapeDtypeStruct((N, D), jnp.float32),
    )(accf, deg[0], h, a2, m, W_l_f, b_l_f.reshape(1, D), W_r_f)

    return out
